# baseline (speedup 1.0000x reference)
import jax
import jax.numpy as jnp
from jax.experimental import pallas as pl
from jax.experimental.pallas import tpu as pltpu

_NJ = 16
_BOOT_CHUNKS = 16


def _mm_body(lhs_any, feed_ref, rhs_ref, out_ref, s0, s1, staging, boot_sem):
    i = pl.program_id(0)
    j = pl.program_id(1)
    half, K = s0.shape
    bk = K // _BOOT_CHUNKS

    @pl.when(jnp.logical_and(i == 0, j == 0))
    def _():
        def copy(k, slot):
            return pltpu.make_async_copy(
                lhs_any.at[pl.ds(0, half), pl.ds(k * bk, bk)],
                staging.at[slot],
                boot_sem.at[slot],
            )

        copy(0, 0).start()

        def boot_body(k, _):
            cur = jax.lax.rem(k, 2)

            @pl.when(k + 1 < _BOOT_CHUNKS)
            def _():
                copy(k + 1, jax.lax.rem(k + 1, 2)).start()

            copy(k, cur).wait()
            s0[:, pl.ds(k * bk, bk)] = staging[cur].astype(jnp.bfloat16)
            return 0

        jax.lax.fori_loop(0, _BOOT_CHUNKS, boot_body, 0)

    @pl.when(i == 0)
    def _():
        bn = feed_ref.shape[1]
        s1[:, pl.ds(j * bn, bn)] = feed_ref[...].astype(jnp.bfloat16)

    @pl.when(i == 0)
    def _():
        out_ref[...] = jnp.dot(s0[...], rhs_ref[...].astype(jnp.bfloat16),
                               preferred_element_type=jnp.float32)

    @pl.when(i == 1)
    def _():
        out_ref[...] = jnp.dot(s1[...], rhs_ref[...].astype(jnp.bfloat16),
                               preferred_element_type=jnp.float32)


def kernel(lhs, rhs):
    M, K = lhs.shape
    _, N = rhs.shape
    half = M // 2
    bn = N // _NJ

    cost = pl.CostEstimate(
        flops=2 * M * N * K,
        transcendentals=0,
        bytes_accessed=(M * K + 2 * K * N + M * N) * 4,
    )
    return pl.pallas_call(
        _mm_body,
        out_shape=jax.ShapeDtypeStruct((M, N), jnp.float32),
        grid=(2, _NJ),
        in_specs=[
            pl.BlockSpec(memory_space=pl.ANY),
            pl.BlockSpec((half, bn),
                         lambda i, j: (1, jnp.where(i == 0, j, _NJ - 1))),
            pl.BlockSpec((K, bn), lambda i, j: (0, j)),
        ],
        out_specs=pl.BlockSpec((half, bn), lambda i, j: (i, j)),
        scratch_shapes=[
            pltpu.VMEM((half, K), jnp.bfloat16),
            pltpu.VMEM((half, K), jnp.bfloat16),
            pltpu.VMEM((2, half, K // _BOOT_CHUNKS), jnp.float32),

            pltpu.SemaphoreType.DMA((2,)),
        ],
        compiler_params=pltpu.CompilerParams(
            dimension_semantics=("arbitrary", "arbitrary"),
            vmem_limit_bytes=63 * 1024 * 1024,
        ),
        cost_estimate=cost,
    )(lhs, lhs, rhs)


# speedup vs baseline: 1.0683x; 1.0683x over previous
import jax
import jax.numpy as jnp
from jax.experimental import pallas as pl
from jax.experimental.pallas import tpu as pltpu

_BM = 1024
_BN = 512


def _mm_body(lhs_ref, rhs_ref, out_ref):
    out_ref[...] = jnp.dot(
        lhs_ref[...].astype(jnp.bfloat16),
        rhs_ref[...].astype(jnp.bfloat16),
        preferred_element_type=jnp.float32,
    )


def kernel(lhs, rhs):
    M, K = lhs.shape
    _, N = rhs.shape

    nj = N // _BN

    def _serp(i, j):
        return jnp.where(i % 2 == 0, j, nj - 1 - j)

    grid = (M // _BM, nj)
    cost = pl.CostEstimate(
        flops=2 * M * N * K,
        transcendentals=0,
        bytes_accessed=(M * K + K * N + M * N) * 4,
    )
    return pl.pallas_call(
        _mm_body,
        out_shape=jax.ShapeDtypeStruct((M, N), jnp.float32),
        grid=grid,
        in_specs=[
            pl.BlockSpec((_BM, K), lambda i, j: (i, 0)),
            pl.BlockSpec((K, _BN), lambda i, j: (0, _serp(i, j))),
        ],
        out_specs=pl.BlockSpec((_BM, _BN), lambda i, j: (i, _serp(i, j))),
        compiler_params=pltpu.CompilerParams(
            dimension_semantics=("arbitrary", "arbitrary"),
            vmem_limit_bytes=60 * 1024 * 1024,
        ),
        cost_estimate=cost,
    )(lhs, rhs)
